# baseline (speedup 1.0000x reference)
import math

import jax
import jax.numpy as jnp
import numpy as np
from jax import lax
from jax.experimental import pallas as pl
from jax.experimental.pallas import tpu as pltpu
from jax.experimental.pallas import tpu_sc as plsc

_N = 256
_ROWS = 32768
_NC = 2
_NS = 16
_NW = _NC * _NS
_TILE_ROWS = 32
_TILES_PER_W = _ROWS // (_NW * _TILE_ROWS)
_TILE_ELEMS = _TILE_ROWS * _N

_W16_RE = np.cos(-2.0 * math.pi * np.arange(16) / 16.0)
_W16_IM = np.sin(-2.0 * math.pi * np.arange(16) / 16.0)


def _br4(x):
    return ((x & 1) << 3) | ((x & 2) << 1) | ((x & 4) >> 1) | ((x & 8) >> 3)


def _fft16_regs(re, im):
    re = [re[_br4(a)] for a in range(16)]
    im = [im[_br4(a)] for a in range(16)]
    for s in range(4):
        stride = 1 << s
        tw_step = 8 >> s
        for g in range(0, 16, 2 * stride):
            for k in range(stride):
                p1 = g + k
                p2 = p1 + stride
                t = k * tw_step
                ar, ai = re[p1], im[p1]
                br, bi = re[p2], im[p2]
                if t == 0:
                    re[p1], im[p1] = ar + br, ai + bi
                    re[p2], im[p2] = ar - br, ai - bi
                elif t == 4:
                    re[p1], im[p1] = ar + bi, ai - br
                    re[p2], im[p2] = ar - bi, ai + br
                else:
                    wr = float(_W16_RE[t])
                    wi = float(_W16_IM[t])
                    wbr = br * wr - bi * wi
                    wbi = br * wi + bi * wr
                    re[p1], im[p1] = ar + wbr, ai + wbi
                    re[p2], im[p2] = ar - wbr, ai - wbi
    return re, im


def _host_tables():
    b = np.arange(16)
    tw_r = np.empty((16, 16), np.float32)
    tw_i = np.empty((16, 16), np.float32)
    for c in range(16):
        ang = -2.0 * math.pi * b * c / 256.0
        tw_r[c] = np.cos(ang)
        tw_i[c] = np.sin(ang)
    sc_idx = np.empty((16, 16), np.int32)
    gt_idx = np.empty((16, 16), np.int32)
    for c in range(16):
        sc_idx[c] = b * 16 + ((c + b) & 15)
    for bb in range(16):
        gt_idx[bb] = bb * 16 + ((bb + b) & 15)
    idx = np.concatenate([sc_idx.reshape(-1), gt_idx.reshape(-1)])
    return tw_r.reshape(-1), tw_i.reshape(-1), idx


def _sc_body(xr, xi, twr_h, twi_h, idx_h, yr, yi,
             inr0, ini0, inr1, ini1, outr0, outi0, outr1, outi1,
             tbr, tbi, tbr2, tbi2, twr, twi, idxv,
             s_ir0, s_ii0, s_ir1, s_ii1, s_or0, s_oi0, s_or1, s_oi1):
    c = lax.axis_index("c")
    s = lax.axis_index("s")
    wid = s * _NC + c
    pltpu.sync_copy(twr_h, twr)
    pltpu.sync_copy(twi_h, twi)
    pltpu.sync_copy(idx_h, idxv)

    ins = ((inr0, ini0, s_ir0, s_ii0), (inr1, ini1, s_ir1, s_ii1))
    outs = ((outr0, outi0, s_or0, s_oi0), (outr1, outi1, s_or1, s_oi1))

    def base_of(t):
        return (wid * _TILES_PER_W + t) * _TILE_ELEMS

    def issue_in(t, b):
        @pl.when(t < _TILES_PER_W)
        def _():
            base = base_of(t)
            pltpu.async_copy(xr.at[pl.ds(base, _TILE_ELEMS)], b[0], b[2])
            pltpu.async_copy(xi.at[pl.ds(base, _TILE_ELEMS)], b[1], b[3])

    def wait_in(b):
        pltpu.make_async_copy(xr.at[pl.ds(0, _TILE_ELEMS)], b[0], b[2]).wait()
        pltpu.make_async_copy(xi.at[pl.ds(0, _TILE_ELEMS)], b[1], b[3]).wait()

    def drain_out(b):
        pltpu.make_async_copy(b[0], yr.at[pl.ds(0, _TILE_ELEMS)], b[2]).wait()
        pltpu.make_async_copy(b[1], yi.at[pl.ds(0, _TILE_ELEMS)], b[3]).wait()

    iota = lax.iota(jnp.int32, 16)
    i16v = iota * 16

    def phase_a(inr, ini, rbase):
        re = [inr[pl.ds(rbase + j * 16, 16)] for j in range(16)]
        im = [ini[pl.ds(rbase + j * 16, 16)] for j in range(16)]
        re, im = _fft16_regs(re, im)
        for cc in range(1, 16):
            wr = twr[pl.ds(cc * 16, 16)]
            wi = twi[pl.ds(cc * 16, 16)]
            tr = re[cc] * wr - im[cc] * wi
            ti = re[cc] * wi + im[cc] * wr
            re[cc], im[cc] = tr, ti
        tbr_s = tbr.at[pl.ds(rbase, _N)]
        tbi_s = tbi.at[pl.ds(rbase, _N)]
        for cc in range(16):
            sidx = idxv[pl.ds(cc * 16, 16)]
            plsc.store_scatter(tbr_s, [sidx], re[cc])
            plsc.store_scatter(tbi_s, [sidx], im[cc])

    def phase_b(outr, outi, rbase):
        re2 = []
        im2 = []
        tbr_s = tbr.at[pl.ds(rbase, _N)]
        tbi_s = tbi.at[pl.ds(rbase, _N)]
        for bb in range(16):
            gidx = idxv[pl.ds(256 + bb * 16, 16)]
            re2.append(plsc.load_gather(tbr_s, [gidx]))
            im2.append(plsc.load_gather(tbi_s, [gidx]))
        re2, im2 = _fft16_regs(re2, im2)
        for d in range(16):
            outr[pl.ds(rbase + d * 16, 16)] = re2[d]
            outi[pl.ds(rbase + d * 16, 16)] = im2[d]

    def process(t, par):
        b = ins[par]
        ob = outs[par]
        wait_in(b)
        issue_in(t + 1, ins[1 - par])

        @pl.when(t >= 2)
        def _():
            drain_out(ob)

        def row_a(r, cr):
            phase_a(b[0], b[1], r * _N)
            return cr

        def row_b(r, cr):
            phase_b(ob[0], ob[1], r * _N)
            return cr

        lax.fori_loop(0, _TILE_ROWS, row_a, 0)
        lax.fori_loop(0, _TILE_ROWS, row_b, 0)
        base = base_of(t)
        pltpu.async_copy(ob[0], yr.at[pl.ds(base, _TILE_ELEMS)], ob[2])
        pltpu.async_copy(ob[1], yi.at[pl.ds(base, _TILE_ELEMS)], ob[3])

    base0 = base_of(0)
    pltpu.async_copy(xr.at[pl.ds(base0, _TILE_ELEMS)], ins[0][0], ins[0][2])
    pltpu.async_copy(xi.at[pl.ds(base0, _TILE_ELEMS)], ins[0][1], ins[0][3])

    def pair(tt, carry):
        t0 = tt * 2
        process(t0, 0)
        process(t0 + 1, 1)
        return carry

    lax.fori_loop(0, _TILES_PER_W // 2, pair, 0)
    drain_out(outs[0])
    drain_out(outs[1])


def kernel(x_re, x_im):
    tw_r, tw_i, idx = _host_tables()
    mesh = plsc.VectorSubcoreMesh(core_axis_name="c", subcore_axis_name="s")
    f = pl.kernel(
        _sc_body,
        out_type=[
            jax.ShapeDtypeStruct((_ROWS * _N,), jnp.float32),
            jax.ShapeDtypeStruct((_ROWS * _N,), jnp.float32),
        ],
        mesh=mesh,
        compiler_params=pltpu.CompilerParams(needs_layout_passes=False),
        scratch_types=[
            pltpu.VMEM((_TILE_ELEMS,), jnp.float32),
            pltpu.VMEM((_TILE_ELEMS,), jnp.float32),
            pltpu.VMEM((_TILE_ELEMS,), jnp.float32),
            pltpu.VMEM((_TILE_ELEMS,), jnp.float32),
            pltpu.VMEM((_TILE_ELEMS,), jnp.float32),
            pltpu.VMEM((_TILE_ELEMS,), jnp.float32),
            pltpu.VMEM((_TILE_ELEMS,), jnp.float32),
            pltpu.VMEM((_TILE_ELEMS,), jnp.float32),
            pltpu.VMEM((_TILE_ELEMS,), jnp.float32),
            pltpu.VMEM((_TILE_ELEMS,), jnp.float32),
            pltpu.VMEM((_N,), jnp.float32),
            pltpu.VMEM((_N,), jnp.float32),
            pltpu.VMEM((_N,), jnp.float32),
            pltpu.VMEM((_N,), jnp.float32),
            pltpu.VMEM((2 * _N,), jnp.int32),
            pltpu.SemaphoreType.DMA,
            pltpu.SemaphoreType.DMA,
            pltpu.SemaphoreType.DMA,
            pltpu.SemaphoreType.DMA,
            pltpu.SemaphoreType.DMA,
            pltpu.SemaphoreType.DMA,
            pltpu.SemaphoreType.DMA,
            pltpu.SemaphoreType.DMA,
        ],
    )
    yr, yi = f(x_re.reshape(-1), x_im.reshape(-1),
               jnp.asarray(tw_r), jnp.asarray(tw_i), jnp.asarray(idx))
    return (yr.reshape(_ROWS, _N), yi.reshape(_ROWS, _N))


# speedup vs baseline: 1.0131x; 1.0131x over previous
import math

import jax
import jax.numpy as jnp
import numpy as np
from jax import lax
from jax.experimental import pallas as pl
from jax.experimental.pallas import tpu as pltpu
from jax.experimental.pallas import tpu_sc as plsc

_N = 256
_ROWS = 32768
_NC = 2
_NS = 16
_NW = _NC * _NS
_TILE_ROWS = 32
_TILES_PER_W = _ROWS // (_NW * _TILE_ROWS)
_TILE_ELEMS = _TILE_ROWS * _N

_W16_RE = np.cos(-2.0 * math.pi * np.arange(16) / 16.0)
_W16_IM = np.sin(-2.0 * math.pi * np.arange(16) / 16.0)


def _br4(x):
    return ((x & 1) << 3) | ((x & 2) << 1) | ((x & 4) >> 1) | ((x & 8) >> 3)


def _fft16_regs(re, im):
    re = [re[_br4(a)] for a in range(16)]
    im = [im[_br4(a)] for a in range(16)]
    for s in range(4):
        stride = 1 << s
        tw_step = 8 >> s
        for g in range(0, 16, 2 * stride):
            for k in range(stride):
                p1 = g + k
                p2 = p1 + stride
                t = k * tw_step
                ar, ai = re[p1], im[p1]
                br, bi = re[p2], im[p2]
                if t == 0:
                    re[p1], im[p1] = ar + br, ai + bi
                    re[p2], im[p2] = ar - br, ai - bi
                elif t == 4:
                    re[p1], im[p1] = ar + bi, ai - br
                    re[p2], im[p2] = ar - bi, ai + br
                else:
                    wr = float(_W16_RE[t])
                    wi = float(_W16_IM[t])
                    wbr = br * wr - bi * wi
                    wbi = br * wi + bi * wr
                    re[p1], im[p1] = ar + wbr, ai + wbi
                    re[p2], im[p2] = ar - wbr, ai - wbi
    return re, im


def _host_tables():
    b = np.arange(16)
    tw_r = np.empty((16, 16), np.float32)
    tw_i = np.empty((16, 16), np.float32)
    for c in range(16):
        ang = -2.0 * math.pi * b * c / 256.0
        tw_r[c] = np.cos(ang)
        tw_i[c] = np.sin(ang)
    sc_idx = np.empty((16, 16), np.int32)
    gt_idx = np.empty((16, 16), np.int32)
    for c in range(16):
        sc_idx[c] = b * 16 + ((c + b) & 15)
    for bb in range(16):
        gt_idx[bb] = bb * 16 + ((bb + b) & 15)
    idx = np.concatenate([sc_idx.reshape(-1), gt_idx.reshape(-1)])
    return tw_r.reshape(-1), tw_i.reshape(-1), idx


def _sc_body(xr, xi, twr_h, twi_h, idx_h, yr, yi,
             inr0, ini0, inr1, ini1, outr0, outi0, outr1, outi1,
             tbr, tbi, tbr2, tbi2, twr, twi, idxv,
             s_ir0, s_ii0, s_ir1, s_ii1, s_or0, s_oi0, s_or1, s_oi1):
    c = lax.axis_index("c")
    s = lax.axis_index("s")
    wid = s * _NC + c
    pltpu.sync_copy(twr_h, twr)
    pltpu.sync_copy(twi_h, twi)
    pltpu.sync_copy(idx_h, idxv)

    ins = ((inr0, ini0, s_ir0, s_ii0), (inr1, ini1, s_ir1, s_ii1))
    outs = ((outr0, outi0, s_or0, s_oi0), (outr1, outi1, s_or1, s_oi1))

    def base_of(t):
        return (wid * _TILES_PER_W + t) * _TILE_ELEMS

    def issue_in(t, b):
        @pl.when(t < _TILES_PER_W)
        def _():
            base = base_of(t)
            pltpu.async_copy(xr.at[pl.ds(base, _TILE_ELEMS)], b[0], b[2])
            pltpu.async_copy(xi.at[pl.ds(base, _TILE_ELEMS)], b[1], b[3])

    def wait_in(b):
        pltpu.make_async_copy(xr.at[pl.ds(0, _TILE_ELEMS)], b[0], b[2]).wait()
        pltpu.make_async_copy(xi.at[pl.ds(0, _TILE_ELEMS)], b[1], b[3]).wait()

    def drain_out(b):
        pltpu.make_async_copy(b[0], yr.at[pl.ds(0, _TILE_ELEMS)], b[2]).wait()
        pltpu.make_async_copy(b[1], yi.at[pl.ds(0, _TILE_ELEMS)], b[3]).wait()

    iota = lax.iota(jnp.int32, 16)
    i16v = iota * 16

    def phase_a(inr, ini, rbase):
        re = [inr[pl.ds(rbase + j * 16, 16)] for j in range(16)]
        im = [ini[pl.ds(rbase + j * 16, 16)] for j in range(16)]
        re, im = _fft16_regs(re, im)
        for cc in range(1, 16):
            wr = twr[pl.ds(cc * 16, 16)]
            wi = twi[pl.ds(cc * 16, 16)]
            tr = re[cc] * wr - im[cc] * wi
            ti = re[cc] * wi + im[cc] * wr
            re[cc], im[cc] = tr, ti
        sb = i16v + rbase
        for cc in range(16):
            sidx = sb + ((iota + cc) & 15)
            plsc.store_scatter(tbr, [sidx], re[cc])
            plsc.store_scatter(tbi, [sidx], im[cc])

    def phase_b(outr, outi, rbase):
        re2 = []
        im2 = []
        for bb in range(16):
            gidx = ((iota + bb) & 15) + (rbase + bb * 16)
            re2.append(plsc.load_gather(tbr, [gidx]))
            im2.append(plsc.load_gather(tbi, [gidx]))
        re2, im2 = _fft16_regs(re2, im2)
        for d in range(16):
            outr[pl.ds(rbase + d * 16, 16)] = re2[d]
            outi[pl.ds(rbase + d * 16, 16)] = im2[d]

    def process(t, par):
        b = ins[par]
        ob = outs[par]
        wait_in(b)
        issue_in(t + 1, ins[1 - par])

        @pl.when(t >= 2)
        def _():
            drain_out(ob)

        @plsc.parallel_loop(0, _TILE_ROWS, unroll=2)
        def _row_a(r):
            phase_a(b[0], b[1], r * _N)

        @plsc.parallel_loop(0, _TILE_ROWS, unroll=2)
        def _row_b(r):
            phase_b(ob[0], ob[1], r * _N)
        base = base_of(t)
        pltpu.async_copy(ob[0], yr.at[pl.ds(base, _TILE_ELEMS)], ob[2])
        pltpu.async_copy(ob[1], yi.at[pl.ds(base, _TILE_ELEMS)], ob[3])

    base0 = base_of(0)
    pltpu.async_copy(xr.at[pl.ds(base0, _TILE_ELEMS)], ins[0][0], ins[0][2])
    pltpu.async_copy(xi.at[pl.ds(base0, _TILE_ELEMS)], ins[0][1], ins[0][3])

    def pair(tt, carry):
        t0 = tt * 2
        process(t0, 0)
        process(t0 + 1, 1)
        return carry

    lax.fori_loop(0, _TILES_PER_W // 2, pair, 0)
    drain_out(outs[0])
    drain_out(outs[1])


def kernel(x_re, x_im):
    tw_r, tw_i, idx = _host_tables()
    mesh = plsc.VectorSubcoreMesh(core_axis_name="c", subcore_axis_name="s")
    f = pl.kernel(
        _sc_body,
        out_type=[
            jax.ShapeDtypeStruct((_ROWS * _N,), jnp.float32),
            jax.ShapeDtypeStruct((_ROWS * _N,), jnp.float32),
        ],
        mesh=mesh,
        compiler_params=pltpu.CompilerParams(needs_layout_passes=False),
        scratch_types=[
            pltpu.VMEM((_TILE_ELEMS,), jnp.float32),
            pltpu.VMEM((_TILE_ELEMS,), jnp.float32),
            pltpu.VMEM((_TILE_ELEMS,), jnp.float32),
            pltpu.VMEM((_TILE_ELEMS,), jnp.float32),
            pltpu.VMEM((_TILE_ELEMS,), jnp.float32),
            pltpu.VMEM((_TILE_ELEMS,), jnp.float32),
            pltpu.VMEM((_TILE_ELEMS,), jnp.float32),
            pltpu.VMEM((_TILE_ELEMS,), jnp.float32),
            pltpu.VMEM((_TILE_ELEMS,), jnp.float32),
            pltpu.VMEM((_TILE_ELEMS,), jnp.float32),
            pltpu.VMEM((_N,), jnp.float32),
            pltpu.VMEM((_N,), jnp.float32),
            pltpu.VMEM((_N,), jnp.float32),
            pltpu.VMEM((_N,), jnp.float32),
            pltpu.VMEM((2 * _N,), jnp.int32),
            pltpu.SemaphoreType.DMA,
            pltpu.SemaphoreType.DMA,
            pltpu.SemaphoreType.DMA,
            pltpu.SemaphoreType.DMA,
            pltpu.SemaphoreType.DMA,
            pltpu.SemaphoreType.DMA,
            pltpu.SemaphoreType.DMA,
            pltpu.SemaphoreType.DMA,
        ],
    )
    yr, yi = f(x_re.reshape(-1), x_im.reshape(-1),
               jnp.asarray(tw_r), jnp.asarray(tw_i), jnp.asarray(idx))
    return (yr.reshape(_ROWS, _N), yi.reshape(_ROWS, _N))


# speedup vs baseline: 1.1448x; 1.1301x over previous
import math

import jax
import jax.numpy as jnp
import numpy as np
from jax import lax
from jax.experimental import pallas as pl
from jax.experimental.pallas import tpu as pltpu
from jax.experimental.pallas import tpu_sc as plsc

_N = 256
_ROWS = 32768
_NC = 2
_NS = 16
_NW = _NC * _NS
_TILE_ROWS = 32
_TILES_PER_W = _ROWS // (_NW * _TILE_ROWS)
_TILE_ELEMS = _TILE_ROWS * _N

_W16_RE = np.cos(-2.0 * math.pi * np.arange(16) / 16.0)
_W16_IM = np.sin(-2.0 * math.pi * np.arange(16) / 16.0)


def _br4(x):
    return ((x & 1) << 3) | ((x & 2) << 1) | ((x & 4) >> 1) | ((x & 8) >> 3)


def _fft16_regs(re, im):
    re = [re[_br4(a)] for a in range(16)]
    im = [im[_br4(a)] for a in range(16)]
    for s in range(4):
        stride = 1 << s
        tw_step = 8 >> s
        for g in range(0, 16, 2 * stride):
            for k in range(stride):
                p1 = g + k
                p2 = p1 + stride
                t = k * tw_step
                ar, ai = re[p1], im[p1]
                br, bi = re[p2], im[p2]
                if t == 0:
                    re[p1], im[p1] = ar + br, ai + bi
                    re[p2], im[p2] = ar - br, ai - bi
                elif t == 4:
                    re[p1], im[p1] = ar + bi, ai - br
                    re[p2], im[p2] = ar - bi, ai + br
                else:
                    wr = float(_W16_RE[t])
                    wi = float(_W16_IM[t])
                    wbr = br * wr - bi * wi
                    wbi = br * wi + bi * wr
                    re[p1], im[p1] = ar + wbr, ai + wbi
                    re[p2], im[p2] = ar - wbr, ai - wbi
    return re, im


def _host_tables():
    b = np.arange(16)
    tw_r = np.empty((16, 16), np.float32)
    tw_i = np.empty((16, 16), np.float32)
    for c in range(16):
        ang = -2.0 * math.pi * b * c / 256.0
        tw_r[c] = np.cos(ang)
        tw_i[c] = np.sin(ang)
    sc_idx = np.empty((16, 16), np.int32)
    gt_idx = np.empty((16, 16), np.int32)
    for c in range(16):
        sc_idx[c] = b * 16 + ((c + b) & 15)
    for bb in range(16):
        gt_idx[bb] = bb * 16 + ((bb + b) & 15)
    idx = np.concatenate([sc_idx.reshape(-1), gt_idx.reshape(-1)])
    return tw_r.reshape(-1), tw_i.reshape(-1), idx


def _sc_body(xr, xi, twr_h, twi_h, idx_h, yr, yi,
             inr0, ini0, inr1, ini1, outr0, outi0, outr1, outi1,
             tbr, tbi, tbr2, tbi2, twr, twi, idxv,
             s_ir0, s_ii0, s_ir1, s_ii1, s_or0, s_oi0, s_or1, s_oi1):
    c = lax.axis_index("c")
    s = lax.axis_index("s")
    wid = s * _NC + c
    pltpu.sync_copy(twr_h, twr)
    pltpu.sync_copy(twi_h, twi)
    pltpu.sync_copy(idx_h, idxv)

    ins = ((inr0, ini0, s_ir0, s_ii0), (inr1, ini1, s_ir1, s_ii1))
    outs = ((outr0, outi0, s_or0, s_oi0), (outr1, outi1, s_or1, s_oi1))

    def base_of(t):
        return (wid * _TILES_PER_W + t) * _TILE_ELEMS

    def issue_in(t, b):
        @pl.when(t < _TILES_PER_W)
        def _():
            base = base_of(t)
            pltpu.async_copy(xr.at[pl.ds(base, _TILE_ELEMS)], b[0], b[2])
            pltpu.async_copy(xi.at[pl.ds(base, _TILE_ELEMS)], b[1], b[3])

    def wait_in(b):
        pltpu.make_async_copy(xr.at[pl.ds(0, _TILE_ELEMS)], b[0], b[2]).wait()
        pltpu.make_async_copy(xi.at[pl.ds(0, _TILE_ELEMS)], b[1], b[3]).wait()

    def drain_out(b):
        pltpu.make_async_copy(b[0], yr.at[pl.ds(0, _TILE_ELEMS)], b[2]).wait()
        pltpu.make_async_copy(b[1], yi.at[pl.ds(0, _TILE_ELEMS)], b[3]).wait()

    iota = lax.iota(jnp.int32, 16)
    i16v = iota * 16

    def phase_a(inr, ini, rbase):
        re = [inr[pl.ds(rbase + j * 16, 16)] for j in range(16)]
        im = [ini[pl.ds(rbase + j * 16, 16)] for j in range(16)]
        re, im = _fft16_regs(re, im)
        for cc in range(1, 16):
            wr = twr[pl.ds(cc * 16, 16)]
            wi = twi[pl.ds(cc * 16, 16)]
            tr = re[cc] * wr - im[cc] * wi
            ti = re[cc] * wi + im[cc] * wr
            re[cc], im[cc] = tr, ti
        sb = i16v + rbase
        for cc in range(16):
            sidx = sb + ((iota + cc) & 15)
            plsc.store_scatter(tbr, [sidx], re[cc])
            plsc.store_scatter(tbi, [sidx], im[cc])

    def phase_b(outr, outi, rbase):
        re2 = []
        im2 = []
        for bb in range(16):
            gidx = ((iota + bb) & 15) + (rbase + bb * 16)
            re2.append(plsc.load_gather(tbr, [gidx]))
            im2.append(plsc.load_gather(tbi, [gidx]))
        re2, im2 = _fft16_regs(re2, im2)
        for d in range(16):
            outr[pl.ds(rbase + d * 16, 16)] = re2[d]
            outi[pl.ds(rbase + d * 16, 16)] = im2[d]

    def process(t, par):
        b = ins[par]
        ob = outs[par]
        wait_in(b)
        issue_in(t + 1, ins[1 - par])

        @pl.when(t >= 2)
        def _():
            drain_out(ob)

        @plsc.parallel_loop(0, _TILE_ROWS, unroll=1)
        def _row_a(r):
            phase_a(b[0], b[1], r * _N)

        @plsc.parallel_loop(0, _TILE_ROWS, unroll=1)
        def _row_b(r):
            phase_b(ob[0], ob[1], r * _N)
        base = base_of(t)
        pltpu.async_copy(ob[0], yr.at[pl.ds(base, _TILE_ELEMS)], ob[2])
        pltpu.async_copy(ob[1], yi.at[pl.ds(base, _TILE_ELEMS)], ob[3])

    base0 = base_of(0)
    pltpu.async_copy(xr.at[pl.ds(base0, _TILE_ELEMS)], ins[0][0], ins[0][2])
    pltpu.async_copy(xi.at[pl.ds(base0, _TILE_ELEMS)], ins[0][1], ins[0][3])

    def pair(tt, carry):
        t0 = tt * 2
        process(t0, 0)
        process(t0 + 1, 1)
        return carry

    lax.fori_loop(0, _TILES_PER_W // 2, pair, 0)
    drain_out(outs[0])
    drain_out(outs[1])


def kernel(x_re, x_im):
    tw_r, tw_i, idx = _host_tables()
    mesh = plsc.VectorSubcoreMesh(core_axis_name="c", subcore_axis_name="s")
    f = pl.kernel(
        _sc_body,
        out_type=[
            jax.ShapeDtypeStruct((_ROWS * _N,), jnp.float32),
            jax.ShapeDtypeStruct((_ROWS * _N,), jnp.float32),
        ],
        mesh=mesh,
        compiler_params=pltpu.CompilerParams(needs_layout_passes=False),
        scratch_types=[
            pltpu.VMEM((_TILE_ELEMS,), jnp.float32),
            pltpu.VMEM((_TILE_ELEMS,), jnp.float32),
            pltpu.VMEM((_TILE_ELEMS,), jnp.float32),
            pltpu.VMEM((_TILE_ELEMS,), jnp.float32),
            pltpu.VMEM((_TILE_ELEMS,), jnp.float32),
            pltpu.VMEM((_TILE_ELEMS,), jnp.float32),
            pltpu.VMEM((_TILE_ELEMS,), jnp.float32),
            pltpu.VMEM((_TILE_ELEMS,), jnp.float32),
            pltpu.VMEM((_TILE_ELEMS,), jnp.float32),
            pltpu.VMEM((_TILE_ELEMS,), jnp.float32),
            pltpu.VMEM((_N,), jnp.float32),
            pltpu.VMEM((_N,), jnp.float32),
            pltpu.VMEM((_N,), jnp.float32),
            pltpu.VMEM((_N,), jnp.float32),
            pltpu.VMEM((2 * _N,), jnp.int32),
            pltpu.SemaphoreType.DMA,
            pltpu.SemaphoreType.DMA,
            pltpu.SemaphoreType.DMA,
            pltpu.SemaphoreType.DMA,
            pltpu.SemaphoreType.DMA,
            pltpu.SemaphoreType.DMA,
            pltpu.SemaphoreType.DMA,
            pltpu.SemaphoreType.DMA,
        ],
    )
    yr, yi = f(x_re.reshape(-1), x_im.reshape(-1),
               jnp.asarray(tw_r), jnp.asarray(tw_i), jnp.asarray(idx))
    return (yr.reshape(_ROWS, _N), yi.reshape(_ROWS, _N))


# speedup vs baseline: 1.1467x; 1.0016x over previous
import math

import jax
import jax.numpy as jnp
import numpy as np
from jax import lax
from jax.experimental import pallas as pl
from jax.experimental.pallas import tpu as pltpu
from jax.experimental.pallas import tpu_sc as plsc

_N = 256
_ROWS = 32768
_NC = 2
_NS = 16
_NW = _NC * _NS
_TILE_ROWS = 32
_TILES_PER_W = _ROWS // (_NW * _TILE_ROWS)
_TILE_ELEMS = _TILE_ROWS * _N

_W16_RE = np.cos(-2.0 * math.pi * np.arange(16) / 16.0)
_W16_IM = np.sin(-2.0 * math.pi * np.arange(16) / 16.0)


def _br4(x):
    return ((x & 1) << 3) | ((x & 2) << 1) | ((x & 4) >> 1) | ((x & 8) >> 3)


def _fft16_regs(re, im):
    re = [re[_br4(a)] for a in range(16)]
    im = [im[_br4(a)] for a in range(16)]
    for s in range(4):
        stride = 1 << s
        tw_step = 8 >> s
        for g in range(0, 16, 2 * stride):
            for k in range(stride):
                p1 = g + k
                p2 = p1 + stride
                t = k * tw_step
                ar, ai = re[p1], im[p1]
                br, bi = re[p2], im[p2]
                if t == 0:
                    re[p1], im[p1] = ar + br, ai + bi
                    re[p2], im[p2] = ar - br, ai - bi
                elif t == 4:
                    re[p1], im[p1] = ar + bi, ai - br
                    re[p2], im[p2] = ar - bi, ai + br
                else:
                    wr = float(_W16_RE[t])
                    wi = float(_W16_IM[t])
                    wbr = br * wr - bi * wi
                    wbi = br * wi + bi * wr
                    re[p1], im[p1] = ar + wbr, ai + wbi
                    re[p2], im[p2] = ar - wbr, ai - wbi
    return re, im


def _host_tables():
    b = np.arange(16)
    tw_r = np.empty((16, 16), np.float32)
    tw_i = np.empty((16, 16), np.float32)
    for c in range(16):
        ang = -2.0 * math.pi * b * c / 256.0
        tw_r[c] = np.cos(ang)
        tw_i[c] = np.sin(ang)
    return tw_r.reshape(-1), tw_i.reshape(-1)


def _sc_body(xr, xi, twr_h, twi_h, yr, yi,
             inr0, ini0, inr1, ini1, outr0, outi0, outr1, outi1,
             tbr, tbi, twr, twi,
             s_ir0, s_ii0, s_ir1, s_ii1, s_or0, s_oi0, s_or1, s_oi1):
    c = lax.axis_index("c")
    s = lax.axis_index("s")
    wid = s * _NC + c
    pltpu.sync_copy(twr_h, twr)
    pltpu.sync_copy(twi_h, twi)

    ins = ((inr0, ini0, s_ir0, s_ii0), (inr1, ini1, s_ir1, s_ii1))
    outs = ((outr0, outi0, s_or0, s_oi0), (outr1, outi1, s_or1, s_oi1))

    def base_of(t):
        return (wid * _TILES_PER_W + t) * _TILE_ELEMS

    def issue_in(t, b):
        @pl.when(t < _TILES_PER_W)
        def _():
            base = base_of(t)
            pltpu.async_copy(xr.at[pl.ds(base, _TILE_ELEMS)], b[0], b[2])
            pltpu.async_copy(xi.at[pl.ds(base, _TILE_ELEMS)], b[1], b[3])

    def wait_in(b):
        pltpu.make_async_copy(xr.at[pl.ds(0, _TILE_ELEMS)], b[0], b[2]).wait()
        pltpu.make_async_copy(xi.at[pl.ds(0, _TILE_ELEMS)], b[1], b[3]).wait()

    def drain_out(b):
        pltpu.make_async_copy(b[0], yr.at[pl.ds(0, _TILE_ELEMS)], b[2]).wait()
        pltpu.make_async_copy(b[1], yi.at[pl.ds(0, _TILE_ELEMS)], b[3]).wait()

    iota = lax.iota(jnp.int32, 16)
    i16v = iota * 16

    def phase_a(inr, ini, rbase):
        re = [inr[pl.ds(rbase + j * 16, 16)] for j in range(16)]
        im = [ini[pl.ds(rbase + j * 16, 16)] for j in range(16)]
        re, im = _fft16_regs(re, im)
        for cc in range(1, 16):
            wr = twr[pl.ds(cc * 16, 16)]
            wi = twi[pl.ds(cc * 16, 16)]
            tr = re[cc] * wr - im[cc] * wi
            ti = re[cc] * wi + im[cc] * wr
            re[cc], im[cc] = tr, ti
        sb = i16v + rbase
        for cc in range(16):
            sidx = sb + ((iota + cc) & 15)
            plsc.store_scatter(tbr, [sidx], re[cc])
            plsc.store_scatter(tbi, [sidx], im[cc])

    def phase_b(outr, outi, rbase):
        re2 = []
        im2 = []
        for bb in range(16):
            gidx = ((iota + bb) & 15) + (rbase + bb * 16)
            re2.append(plsc.load_gather(tbr, [gidx]))
            im2.append(plsc.load_gather(tbi, [gidx]))
        re2, im2 = _fft16_regs(re2, im2)
        for d in range(16):
            outr[pl.ds(rbase + d * 16, 16)] = re2[d]
            outi[pl.ds(rbase + d * 16, 16)] = im2[d]

    def process(t, par):
        b = ins[par]
        ob = outs[par]
        wait_in(b)
        issue_in(t + 1, ins[1 - par])

        @pl.when(t >= 2)
        def _():
            drain_out(ob)

        @plsc.parallel_loop(0, _TILE_ROWS, unroll=1)
        def _row_a(r):
            phase_a(b[0], b[1], r * _N)

        @plsc.parallel_loop(0, _TILE_ROWS, unroll=1)
        def _row_b(r):
            phase_b(ob[0], ob[1], r * _N)
        base = base_of(t)
        pltpu.async_copy(ob[0], yr.at[pl.ds(base, _TILE_ELEMS)], ob[2])
        pltpu.async_copy(ob[1], yi.at[pl.ds(base, _TILE_ELEMS)], ob[3])

    base0 = base_of(0)
    pltpu.async_copy(xr.at[pl.ds(base0, _TILE_ELEMS)], ins[0][0], ins[0][2])
    pltpu.async_copy(xi.at[pl.ds(base0, _TILE_ELEMS)], ins[0][1], ins[0][3])

    def pair(tt, carry):
        t0 = tt * 2
        process(t0, 0)
        process(t0 + 1, 1)
        return carry

    lax.fori_loop(0, _TILES_PER_W // 2, pair, 0)
    drain_out(outs[0])
    drain_out(outs[1])


def kernel(x_re, x_im):
    tw_r, tw_i = _host_tables()
    mesh = plsc.VectorSubcoreMesh(core_axis_name="c", subcore_axis_name="s")
    f = pl.kernel(
        _sc_body,
        out_type=[
            jax.ShapeDtypeStruct((_ROWS * _N,), jnp.float32),
            jax.ShapeDtypeStruct((_ROWS * _N,), jnp.float32),
        ],
        mesh=mesh,
        compiler_params=pltpu.CompilerParams(needs_layout_passes=False),
        scratch_types=[
            pltpu.VMEM((_TILE_ELEMS,), jnp.float32),
            pltpu.VMEM((_TILE_ELEMS,), jnp.float32),
            pltpu.VMEM((_TILE_ELEMS,), jnp.float32),
            pltpu.VMEM((_TILE_ELEMS,), jnp.float32),
            pltpu.VMEM((_TILE_ELEMS,), jnp.float32),
            pltpu.VMEM((_TILE_ELEMS,), jnp.float32),
            pltpu.VMEM((_TILE_ELEMS,), jnp.float32),
            pltpu.VMEM((_TILE_ELEMS,), jnp.float32),
            pltpu.VMEM((_TILE_ELEMS,), jnp.float32),
            pltpu.VMEM((_TILE_ELEMS,), jnp.float32),
            pltpu.VMEM((_N,), jnp.float32),
            pltpu.VMEM((_N,), jnp.float32),
            pltpu.SemaphoreType.DMA,
            pltpu.SemaphoreType.DMA,
            pltpu.SemaphoreType.DMA,
            pltpu.SemaphoreType.DMA,
            pltpu.SemaphoreType.DMA,
            pltpu.SemaphoreType.DMA,
            pltpu.SemaphoreType.DMA,
            pltpu.SemaphoreType.DMA,
        ],
    )
    yr, yi = f(x_re.reshape(-1), x_im.reshape(-1),
               jnp.asarray(tw_r), jnp.asarray(tw_i))
    return (yr.reshape(_ROWS, _N), yi.reshape(_ROWS, _N))
